# Initial kernel scaffold; baseline (speedup 1.0000x reference)
import functools

import jax
import jax.numpy as jnp
from jax import lax
from jax.experimental import pallas as pl
from jax.experimental.pallas import tpu as pltpu
from jax.experimental.pallas import tpu_sc as plsc

_B = 4096
_K = 26
_V = 1000
_D = 64
_N = _B * _K
_NW = 32
_RPW = _N // _NW
_G = 128
_NG = _RPW // _G
_L = 16


def _sc_gather(w_flat, kjt3):
    mesh = plsc.VectorSubcoreMesh(core_axis_name="c", subcore_axis_name="s")

    @functools.partial(
        pl.kernel,
        out_type=jax.ShapeDtypeStruct((_N, _D), jnp.float32),
        mesh=mesh,
        scratch_types=[
            pltpu.VMEM((_NG, _G), jnp.int32),
            pltpu.VMEM((_NG, _G), jnp.int32),
            pltpu.VMEM((_G, _D), jnp.float32),
            pltpu.SemaphoreType.DMA,
        ],
    )
    def k(w_hbm, kjt_hbm, out_hbm, kjt_v, idx_v, rows_v, sem):
        nc = 2
        wid = lax.axis_index("s") * nc + lax.axis_index("c")
        base = wid * _RPW

        pltpu.sync_copy(kjt_hbm.at[wid], kjt_v)

        def compute(j, _):
            t0 = j * _L
            r = t0 // _G
            c = lax.rem(t0, _G)
            tvec = lax.iota(jnp.int32, _L) + t0
            vals = kjt_v[r, pl.ds(c, _L)]
            vals = jnp.clip(vals, 0, _V - 1)
            keys = lax.rem(tvec, _K)
            idx_v[r, pl.ds(c, _L)] = keys * _V + vals
            return 0

        lax.fori_loop(0, _RPW // _L, compute, 0)

        def gather(g, _):
            pltpu.async_copy(w_hbm.at[idx_v.at[g]], rows_v, sem).wait()
            pltpu.sync_copy(rows_v, out_hbm.at[pl.ds(base + g * _G, _G)])
            return 0

        lax.fori_loop(0, _NG, gather, 0)

    return k(w_flat, kjt3)


def kernel(kjt, W):
    w_flat = W.reshape(_K * _V, _D)
    kjt3 = kjt.astype(jnp.int32).reshape(_NW, _NG, _G)
    out = _sc_gather(w_flat, kjt3)
    return out.reshape(_B, _K * _D)


# speedup vs baseline: 17.8830x; 17.8830x over previous
import functools

import jax
import jax.numpy as jnp
from jax import lax
from jax.experimental import pallas as pl
from jax.experimental.pallas import tpu as pltpu
from jax.experimental.pallas import tpu_sc as plsc

_B = 4096
_K = 26
_V = 1000
_D = 64
_N = _B * _K
_NW = 32
_RPW = _N // _NW
_G = 128
_NG = _RPW // _G
_L = 16


def _sc_gather(w_flat, kjt3):
    mesh = plsc.VectorSubcoreMesh(
        core_axis_name="c", subcore_axis_name="s", num_cores=2, num_subcores=16
    )

    @functools.partial(
        pl.kernel,
        out_type=jax.ShapeDtypeStruct((_N, _D), jnp.float32),
        mesh=mesh,
        scratch_types=[
            pltpu.VMEM((_NG, _G), jnp.int32),
            pltpu.VMEM((_NG, _G), jnp.int32),
            pltpu.VMEM((_G, _D), jnp.float32),
            pltpu.SemaphoreType.DMA,
        ],
        compiler_params=pltpu.CompilerParams(use_tc_tiling_on_sc=False),
    )
    def k(w_hbm, kjt_hbm, out_hbm, kjt_v, idx_v, rows_v, sem):
        nc = 2
        wid = lax.axis_index("s") * nc + lax.axis_index("c")
        base = wid * _RPW

        pltpu.sync_copy(kjt_hbm.at[wid], kjt_v)

        def compute(j, _):
            t0 = j * _L
            r = t0 // _G
            c = lax.rem(t0, _G)
            tvec = lax.iota(jnp.int32, _L) + t0
            vals = kjt_v[r, pl.ds(c, _L)]
            vals = jnp.clip(vals, 0, _V - 1)
            keys = lax.rem(tvec, _K)
            idx_v[r, pl.ds(c, _L)] = keys * _V + vals
            return 0

        lax.fori_loop(0, _RPW // _L, compute, 0)

        def gather(g, _):
            pltpu.async_copy(w_hbm.at[idx_v.at[g]], rows_v, sem).wait()
            pltpu.sync_copy(rows_v, out_hbm.at[pl.ds(base + g * _G, _G)])
            return 0

        lax.fori_loop(0, _NG, gather, 0)

    return k(w_flat, kjt3)


def kernel(kjt, W):
    w_flat = W.reshape(_K * _V, _D)
    kjt3 = kjt.astype(jnp.int32).reshape(_NW, _NG, _G)
    out = _sc_gather(w_flat, kjt3)
    return out.reshape(_B, _K * _D)


# speedup vs baseline: 20.5415x; 1.1487x over previous
import functools

import jax
import jax.numpy as jnp
from jax import lax
from jax.experimental import pallas as pl
from jax.experimental.pallas import tpu as pltpu
from jax.experimental.pallas import tpu_sc as plsc

_B = 4096
_K = 26
_V = 1000
_D = 64
_N = _B * _K
_NW = 32
_RPW = _N // _NW
_G = 128
_NG = _RPW // _G
_L = 16


def _sc_gather(w_flat, kjt3):
    mesh = plsc.VectorSubcoreMesh(
        core_axis_name="c", subcore_axis_name="s", num_cores=2, num_subcores=16
    )

    @functools.partial(
        pl.kernel,
        out_type=jax.ShapeDtypeStruct((_N, _D), jnp.float32),
        mesh=mesh,
        scratch_types=[
            pltpu.VMEM((_NG, _G), jnp.int32),
            pltpu.VMEM((_NG, _G), jnp.int32),
            pltpu.VMEM((2, _G, _D), jnp.float32),
            pltpu.SemaphoreType.DMA,
            pltpu.SemaphoreType.DMA,
            pltpu.SemaphoreType.DMA,
            pltpu.SemaphoreType.DMA,
        ],
        compiler_params=pltpu.CompilerParams(use_tc_tiling_on_sc=False),
    )
    def k(w_hbm, kjt_hbm, out_hbm, kjt_v, idx_v, rows_v,
          gsem0, gsem1, wsem0, wsem1):
        nc = 2
        wid = lax.axis_index("s") * nc + lax.axis_index("c")
        base = wid * _RPW
        gsems = (gsem0, gsem1)
        wsems = (wsem0, wsem1)

        pltpu.sync_copy(kjt_hbm.at[wid], kjt_v)

        def compute_row(r):
            def body(j, _):
                t0 = r * _G + j * _L
                tvec = lax.iota(jnp.int32, _L) + t0
                vals = jnp.clip(kjt_v[r, pl.ds(j * _L, _L)], 0, _V - 1)
                idx_v[r, pl.ds(j * _L, _L)] = lax.rem(tvec, _K) * _V + vals
                return 0

            lax.fori_loop(0, _G // _L, body, 0)

        def fire_gather(g, buf):
            return pltpu.async_copy(
                w_hbm.at[idx_v.at[g]], rows_v.at[buf], gsems[buf]
            )

        gc = [None, None]
        wc = [None, None]
        compute_row(0)
        gc[0] = fire_gather(0, 0)
        for g in range(_NG):
            b = g % 2
            nb = 1 - b
            if g + 1 < _NG:
                compute_row(g + 1)
                if wc[nb] is not None:
                    wc[nb].wait()
                gc[nb] = fire_gather(g + 1, nb)
            gc[b].wait()
            wc[b] = pltpu.async_copy(
                rows_v.at[b], out_hbm.at[pl.ds(base + g * _G, _G)], wsems[b]
            )
        wc[0].wait()
        wc[1].wait()

    return k(w_flat, kjt3)


def kernel(kjt, W):
    w_flat = W.reshape(_K * _V, _D)
    kjt3 = kjt.astype(jnp.int32).reshape(_NW, _NG, _G)
    out = _sc_gather(w_flat, kjt3)
    return out.reshape(_B, _K * _D)
